# Initial kernel scaffold; baseline (speedup 1.0000x reference)
import jax
import jax.numpy as jnp
from jax.experimental import pallas as pl


def kernel(hidden_states, router_weight, gate_up_proj, down_proj):
    raise NotImplementedError("write your pallas kernel here")


# speedup vs baseline: 1.1431x; 1.1431x over previous
import jax
import jax.numpy as jnp
from jax import lax
from jax.experimental import pallas as pl
from jax.experimental.pallas import tpu as pltpu

E = 8
TOP_K = 2
D = 768
F = 2048
FC = 512
NC = F // FC


def _moe_body(x32_ref, xb_ref, wr_ref, gate_ref, up_ref, down_ref,
              out_ref, comb_ref):
    e = pl.program_id(0)
    c = pl.program_id(1)

    @pl.when((e == 0) & (c == 0))
    def _router():
        x32 = x32_ref[...]
        logits = lax.dot_general(
            x32, wr_ref[...], (((1,), (1,)), ((), ())),
            preferred_element_type=jnp.float32)
        idx = lax.broadcasted_iota(jnp.int32, logits.shape, 1)
        m1 = jnp.max(logits, axis=1, keepdims=True)
        i1 = jnp.min(jnp.where(logits == m1, idx, E), axis=1, keepdims=True)
        is1 = idx == i1
        masked = jnp.where(is1, -jnp.inf, logits)
        m2 = jnp.max(masked, axis=1, keepdims=True)
        i2 = jnp.min(jnp.where(masked == m2, idx, E), axis=1, keepdims=True)
        is2 = idx == i2
        w1 = 1.0 / (1.0 + jnp.exp(m2 - m1))
        comb_ref[...] = jnp.where(is1, w1, jnp.where(is2, 1.0 - w1, 0.0))
        out_ref[...] = jnp.zeros_like(out_ref)

    xb = xb_ref[...]
    gate = jnp.dot(xb, gate_ref[0], preferred_element_type=jnp.float32)
    up = jnp.dot(xb, up_ref[0], preferred_element_type=jnp.float32)
    act = (gate * jax.nn.sigmoid(gate) * up).astype(jnp.bfloat16)
    y = jnp.dot(act, down_ref[0], preferred_element_type=jnp.float32)
    idx = lax.broadcasted_iota(jnp.int32, (1, E), 1)
    w = jnp.sum(jnp.where(idx == e, comb_ref[...], 0.0), axis=1, keepdims=True)
    out_ref[...] += y * w


def kernel(hidden_states, router_weight, gate_up_proj, down_proj):
    B, S, _ = hidden_states.shape
    T = B * S
    x32 = hidden_states.reshape(T, D)
    xb = x32.astype(jnp.bfloat16)
    gub = gate_up_proj.astype(jnp.bfloat16)
    dnb = down_proj.astype(jnp.bfloat16)

    out = pl.pallas_call(
        _moe_body,
        grid=(E, NC),
        in_specs=[
            pl.BlockSpec((T, D), lambda e, c: (0, 0)),
            pl.BlockSpec((T, D), lambda e, c: (0, 0)),
            pl.BlockSpec((E, D), lambda e, c: (0, 0)),
            pl.BlockSpec((1, D, FC), lambda e, c: (e, 0, c)),
            pl.BlockSpec((1, D, FC), lambda e, c: (e, 0, c + NC)),
            pl.BlockSpec((1, FC, D), lambda e, c: (e, c, 0)),
        ],
        out_specs=pl.BlockSpec((T, D), lambda e, c: (0, 0)),
        out_shape=jax.ShapeDtypeStruct((T, D), jnp.float32),
        scratch_shapes=[pltpu.VMEM((T, E), jnp.float32)],
        compiler_params=pltpu.CompilerParams(
            dimension_semantics=("arbitrary", "arbitrary"),
        ),
    )(x32, xb, router_weight, gub, gub, dnb)
    return out.reshape(B, S, D)
